# baseline (speedup 1.0000x reference)
import functools

import jax
import jax.numpy as jnp
from jax import lax
from jax.experimental import pallas as pl
from jax.experimental.pallas import tpu as pltpu
from jax.experimental.pallas import tpu_sc as plsc

B = 4096
SYN = 32
SEM = 64
HID = 512
OUT = 4096
NNZ = 16384
HV_V = 1000000

NC = 2
NS = 16
LANES = 16
NW = NC * NS
PADW = 128
CHUNK = 128
NZ_PER_SC = NNZ // NC
NZ_PER_W = NZ_PER_SC // NS
NCHUNK = NZ_PER_W // CHUNK
CAT_PER_W = B // NW
ROWS_PER_T = B // NS
BM = 512


def _bcast(vec, j):
    return lax.gather(
        vec, jnp.full((LANES, 1), j, jnp.int32),
        lax.GatherDimensionNumbers(
            offset_dims=(), collapsed_slice_dims=(0,), start_index_map=(0,)),
        (1,), mode=lax.GatherScatterMode.PROMISE_IN_BOUNDS)


def _sc_body(cols_h, rows_h, half_h, vals_h, catix_h, table_h, cattab_h,
             parts_h, catbe_h,
             colv, rowv, hvv, valv, ga, gb, sbuf, catrows,
             acc0, sema, semb):
    core = lax.axis_index("c")
    sid = lax.axis_index("s")
    wid = core * NS + sid
    gbufs = (ga, gb)
    sems = (sema, semb)

    def _zero(i, carry):
        z = jnp.zeros((LANES,), jnp.float32)
        for g in range(PADW // LANES):
            sbuf[i, pl.ds(g * LANES, LANES)] = z
        return carry

    lax.fori_loop(0, CHUNK, _zero, 0)

    for half in range(ROWS_PER_T // CHUNK):
        pltpu.sync_copy(
            sbuf, acc0.at[pl.ds(sid * ROWS_PER_T + half * CHUNK, CHUNK)])

    pltpu.sync_copy(catix_h.at[pl.ds(wid, 1)], colv.at[pl.ds(0, 1)])
    pltpu.async_copy(cattab_h.at[colv.at[0]], catrows, sema).wait()
    pltpu.sync_copy(catrows, catbe_h.at[pl.ds(wid * CAT_PER_W, CAT_PER_W)])

    plsc.subcore_barrier()

    crow0 = core * (NZ_PER_SC // CHUNK) + sid * NCHUNK
    for s in range(3):
        pltpu.sync_copy(cols_h.at[s, pl.ds(crow0, NCHUNK)], colv)
        pltpu.sync_copy(rows_h.at[s, pl.ds(crow0, NCHUNK)], rowv)
        pltpu.sync_copy(half_h.at[s, pl.ds(crow0, NCHUNK)], hvv)
        pltpu.sync_copy(vals_h.at[s, pl.ds(crow0, NCHUNK)], valv)

        cps = [None] * NCHUNK
        cps[0] = pltpu.async_copy(table_h.at[colv.at[0]], gbufs[0], sems[0])
        for k in range(NCHUNK):
            if k + 1 < NCHUNK:
                cps[k + 1] = pltpu.async_copy(
                    table_h.at[colv.at[k + 1]],
                    gbufs[(k + 1) % 2], sems[(k + 1) % 2])
            cps[k].wait()
            buf = gbufs[k % 2]

            def _scale(i, carry, k=k, buf=buf):
                vv = valv[k, pl.ds(i * LANES, LANES)]
                hh = hvv[k, pl.ds(i * LANES, LANES)]
                for j in range(LANES):
                    vj = _bcast(vv, j)
                    hj = _bcast(hh, j)
                    row = i * LANES + j
                    for g in range(SEM // LANES):
                        lo = buf[row, pl.ds(g * LANES, LANES)]
                        hi = buf[row, pl.ds(SEM + g * LANES, LANES)]
                        sbuf[row, pl.ds(g * LANES, LANES)] = (
                            (lo + hj * (hi - lo)) * vj)
                return carry

            lax.fori_loop(0, CHUNK // LANES, _scale, 0)

            pltpu.sync_copy(sbuf, acc0.at[rowv.at[k]], add=True)

        plsc.subcore_barrier()
        rsl = pl.ds(sid * ROWS_PER_T, ROWS_PER_T)
        pltpu.sync_copy(acc0.at[rsl], parts_h.at[core, s, rsl])
        if s < 2:
            for half in range(ROWS_PER_T // CHUNK):
                pltpu.sync_copy(
                    sbuf, acc0.at[pl.ds(sid * ROWS_PER_T + half * CHUNK,
                                        CHUNK)])
            plsc.subcore_barrier()


@functools.cache
def _sc_spmm_prog():
  return functools.partial(
    pl.kernel,
    out_type=(
        jax.ShapeDtypeStruct((NC, 3, B, PADW), jnp.float32),
        jax.ShapeDtypeStruct((B, PADW), jnp.float32),
    ),
    mesh=plsc.VectorSubcoreMesh(
        core_axis_name="c", subcore_axis_name="s",
        num_cores=NC, num_subcores=NS),
    scratch_types=[
        pltpu.VMEM((NCHUNK, CHUNK), jnp.int32),
        pltpu.VMEM((NCHUNK, CHUNK), jnp.int32),
        pltpu.VMEM((NCHUNK, CHUNK), jnp.float32),
        pltpu.VMEM((NCHUNK, CHUNK), jnp.float32),
        pltpu.VMEM((CHUNK, PADW), jnp.float32),
        pltpu.VMEM((CHUNK, PADW), jnp.float32),
        pltpu.VMEM((CHUNK, PADW), jnp.float32),
        pltpu.VMEM((CAT_PER_W, PADW), jnp.float32),
        pltpu.VMEM_SHARED((B, PADW), jnp.float32),
        pltpu.SemaphoreType.DMA,
        pltpu.SemaphoreType.DMA,
    ],
  )(_sc_body)


def _tc_body(scale_ref, cat_ref, parts_ref, topb_ref, topf_ref, topa_ref,
             small_ref, w1c_ref, w1b_ref, w1f_ref, w1a_ref, w1s_ref, b1_ref,
             w2_ref, b2_ref, out_ref):
    scale = scale_ref[0, 0]

    def dg(x, w):
        return lax.dot_general(x, w, (((1,), (1,)), ((), ())),
                               preferred_element_type=jnp.float32)

    def part(s):
        return parts_ref[0, s] + parts_ref[1, s]

    hvb = topb_ref[...] + scale * part(0)
    hvf = topf_ref[...] + scale * part(1)
    hva = topa_ref[...] + scale * part(2)
    h = (dg(cat_ref[...], w1c_ref[...]) + dg(hvb, w1b_ref[...])
         + dg(hvf, w1f_ref[...]) + dg(hva, w1a_ref[...])
         + dg(small_ref[...], w1s_ref[...]) + b1_ref[...])
    h = jnp.maximum(h, 0.0)
    logits = dg(h, w2_ref[...]) + b2_ref[...]
    m = jnp.max(logits, axis=1, keepdims=True)
    lse = jnp.log(jnp.sum(jnp.exp(logits - m), axis=1, keepdims=True)) + m
    out_ref[...] = logits - lse


_tc_mlp = pl.pallas_call(
    _tc_body,
    grid=(B // BM,),
    in_specs=[
        pl.BlockSpec(memory_space=pltpu.SMEM),
        pl.BlockSpec((BM, PADW), lambda i: (i, 0)),
        pl.BlockSpec((NC, 3, BM, PADW), lambda i: (0, 0, i, 0)),
        pl.BlockSpec((BM, PADW), lambda i: (i, 0)),
        pl.BlockSpec((BM, PADW), lambda i: (i, 0)),
        pl.BlockSpec((BM, PADW), lambda i: (i, 0)),
        pl.BlockSpec((BM, 8), lambda i: (i, 0)),
        pl.BlockSpec((HID, PADW), lambda i: (0, 0)),
        pl.BlockSpec((HID, PADW), lambda i: (0, 0)),
        pl.BlockSpec((HID, PADW), lambda i: (0, 0)),
        pl.BlockSpec((HID, PADW), lambda i: (0, 0)),
        pl.BlockSpec((HID, 8), lambda i: (0, 0)),
        pl.BlockSpec((1, HID), lambda i: (0, 0)),
        pl.BlockSpec((OUT, HID), lambda i: (0, 0)),
        pl.BlockSpec((1, OUT), lambda i: (0, 0)),
    ],
    out_specs=pl.BlockSpec((BM, OUT), lambda i: (i, 0)),
    out_shape=jax.ShapeDtypeStruct((B, OUT), jnp.float32),
    compiler_params=pltpu.CompilerParams(
        dimension_semantics=("arbitrary",)),
)


def kernel(d_onehot, cat_b_ix, hvb_rows, hvb_cols, hvb_vals, hvb_top,
           hvf_rows, hvf_cols, hvf_vals, hvf_top,
           hva_rows, hva_cols, hva_vals, hva_top,
           nullA, use_gpu, ablate_sem,
           cat_embeds, hvec_embeds, W1, b1, W2, b2):
    del use_gpu
    i32, f32 = jnp.int32, jnp.float32
    cols = jnp.stack([hvb_cols, hvf_cols, hva_cols]).astype(i32)
    cols3 = (cols // 2).reshape(3, NNZ // CHUNK, CHUNK)
    half3 = (cols % 2).astype(f32).reshape(3, NNZ // CHUNK, CHUNK)
    rows3 = jnp.stack([hvb_rows, hvf_rows, hva_rows]).astype(i32) \
        .reshape(3, NNZ // CHUNK, CHUNK)
    vals3 = jnp.stack([hvb_vals, hvf_vals, hva_vals]).astype(f32) \
        .reshape(3, NNZ // CHUNK, CHUNK)
    catix = cat_b_ix.astype(i32).reshape(NW, CAT_PER_W)
    table2 = hvec_embeds.astype(f32).reshape(HV_V // 2, PADW)
    cattab = jnp.pad(cat_embeds.astype(f32), ((0, 0), (0, PADW - SYN)))

    parts, cat_be = _sc_spmm_prog()(cols3, rows3, half3, vals3, catix,
                                    table2, cattab)

    scale = jnp.where(jnp.asarray(ablate_sem) != 0, 0.0, 1.0) \
        .astype(f32).reshape(1, 1)
    small = jnp.concatenate([nullA[:, None], d_onehot], axis=1).astype(f32)
    w1c = jnp.pad(W1[:, :SYN], ((0, 0), (0, PADW - SYN)))
    def padw(x):
        return jnp.pad(x, ((0, 0), (0, PADW - SEM)))
    w1b = padw(W1[:, SYN:SYN + SEM])
    w1f = padw(W1[:, SYN + SEM:SYN + 2 * SEM])
    w1a = padw(W1[:, SYN + 2 * SEM:SYN + 3 * SEM])
    w1s = W1[:, SYN + 3 * SEM:]
    return _tc_mlp(scale, cat_be, parts,
                   padw(hvb_top), padw(hvf_top), padw(hva_top), small,
                   w1c, w1b, w1f, w1a, w1s,
                   b1.reshape(1, HID), W2, b2.reshape(1, OUT))


# speedup vs baseline: 1.0809x; 1.0809x over previous
import functools

import jax
import jax.numpy as jnp
from jax import lax
from jax.experimental import pallas as pl
from jax.experimental.pallas import tpu as pltpu
from jax.experimental.pallas import tpu_sc as plsc

B = 4096
SYN = 32
SEM = 64
HID = 512
OUT = 4096
NNZ = 16384

NC = 2
NS = 16
LANES = 16
NW = NC * NS
CHUNK = 128
NZ_PER_SC = NNZ // NC
NZ_PER_W = NZ_PER_SC // NS
NCHUNK = NZ_PER_W // CHUNK
CAT_PER_W = B // NW
ROWS_PER_T = B // NS
BM = 512


def _bcast(vec, j):
    return lax.gather(
        vec, jnp.full((LANES, 1), j, jnp.int32),
        lax.GatherDimensionNumbers(
            offset_dims=(), collapsed_slice_dims=(0,), start_index_map=(0,)),
        (1,), mode=lax.GatherScatterMode.PROMISE_IN_BOUNDS)


def _sc_body(cols_h, rows_h, vals_h, catix_h, table_h, cattab_h,
             parts_h, catbe_h,
             colv, rowv, valv, ga, gb, sbuf, catrows,
             acc0, acc1, acc2, sema, semb):
    core = lax.axis_index("c")
    sid = lax.axis_index("s")
    wid = core * NS + sid
    accs = (acc0, acc1, acc2)
    gbufs = (ga, gb)
    sems = (sema, semb)

    def _zero(i, carry):
        z = jnp.zeros((LANES,), jnp.float32)
        for g in range(SEM // LANES):
            sbuf[i, pl.ds(g * LANES, LANES)] = z
        return carry

    lax.fori_loop(0, CHUNK, _zero, 0)

    for s in range(3):
        for half in range(ROWS_PER_T // CHUNK):
            pltpu.sync_copy(
                sbuf,
                accs[s].at[pl.ds(sid * ROWS_PER_T + half * CHUNK, CHUNK)])

    pltpu.sync_copy(catix_h.at[pl.ds(wid, 1)], colv.at[pl.ds(0, 1)])
    pltpu.async_copy(cattab_h.at[colv.at[0]], catrows, sema).wait()
    pltpu.sync_copy(catrows, catbe_h.at[pl.ds(wid * CAT_PER_W, CAT_PER_W)])

    plsc.subcore_barrier()

    crow0 = core * (NZ_PER_SC // CHUNK) + sid * NCHUNK
    for s in range(3):
        pltpu.sync_copy(cols_h.at[s, pl.ds(crow0, NCHUNK)], colv)
        pltpu.sync_copy(rows_h.at[s, pl.ds(crow0, NCHUNK)], rowv)
        pltpu.sync_copy(vals_h.at[s, pl.ds(crow0, NCHUNK)], valv)

        cps = [None] * NCHUNK
        cps[0] = pltpu.async_copy(table_h.at[colv.at[0]], gbufs[0], sems[0])
        for k in range(NCHUNK):
            if k + 1 < NCHUNK:
                cps[k + 1] = pltpu.async_copy(
                    table_h.at[colv.at[k + 1]],
                    gbufs[(k + 1) % 2], sems[(k + 1) % 2])
            cps[k].wait()
            buf = gbufs[k % 2]

            def _scale(i, carry, k=k, buf=buf):
                vv = valv[k, pl.ds(i * LANES, LANES)]
                for j in range(LANES):
                    vj = _bcast(vv, j)
                    row = i * LANES + j
                    for g in range(SEM // LANES):
                        sl = pl.ds(g * LANES, LANES)
                        sbuf[row, sl] = buf[row, sl] * vj
                return carry

            lax.fori_loop(0, CHUNK // LANES, _scale, 0)

            pltpu.sync_copy(sbuf, accs[s].at[rowv.at[k]], add=True)

    plsc.subcore_barrier()
    for s in range(3):
        pltpu.sync_copy(accs[s].at[pl.ds(sid * ROWS_PER_T, ROWS_PER_T)],
                        parts_h.at[core, s, pl.ds(sid * ROWS_PER_T, ROWS_PER_T)])


@functools.cache
def _sc_spmm_prog():
  return functools.partial(
    pl.kernel,
    out_type=(
        jax.ShapeDtypeStruct((NC, 3, B, SEM), jnp.float32),
        jax.ShapeDtypeStruct((B, SYN), jnp.float32),
    ),
    mesh=plsc.VectorSubcoreMesh(
        core_axis_name="c", subcore_axis_name="s",
        num_cores=NC, num_subcores=NS),
    compiler_params=pltpu.CompilerParams(use_tc_tiling_on_sc=False),
    scratch_types=[
        pltpu.VMEM((NCHUNK, CHUNK), jnp.int32),
        pltpu.VMEM((NCHUNK, CHUNK), jnp.int32),
        pltpu.VMEM((NCHUNK, CHUNK), jnp.float32),
        pltpu.VMEM((CHUNK, SEM), jnp.float32),
        pltpu.VMEM((CHUNK, SEM), jnp.float32),
        pltpu.VMEM((CHUNK, SEM), jnp.float32),
        pltpu.VMEM((CAT_PER_W, SYN), jnp.float32),
        pltpu.VMEM_SHARED((B, SEM), jnp.float32),
        pltpu.VMEM_SHARED((B, SEM), jnp.float32),
        pltpu.VMEM_SHARED((B, SEM), jnp.float32),
        pltpu.SemaphoreType.DMA,
        pltpu.SemaphoreType.DMA,
    ],
  )(_sc_body)


def _tc_body(scale_ref, cat_ref, parts_ref, topb_ref, topf_ref, topa_ref,
             small_ref, w1c_ref, w1b_ref, w1f_ref, w1a_ref, w1s_ref, b1_ref,
             w2_ref, b2_ref, out_ref):
    scale = scale_ref[0, 0]

    def dg(x, w):
        return lax.dot_general(x, w, (((1,), (1,)), ((), ())),
                               preferred_element_type=jnp.float32)

    def part(s):
        return parts_ref[0, s] + parts_ref[1, s]

    hvb = topb_ref[...] + scale * part(0)
    hvf = topf_ref[...] + scale * part(1)
    hva = topa_ref[...] + scale * part(2)
    h = (dg(cat_ref[...], w1c_ref[...]) + dg(hvb, w1b_ref[...])
         + dg(hvf, w1f_ref[...]) + dg(hva, w1a_ref[...])
         + dg(small_ref[...], w1s_ref[...]) + b1_ref[...])
    h = jnp.maximum(h, 0.0)
    logits = dg(h, w2_ref[...]) + b2_ref[...]
    m = jnp.max(logits, axis=1, keepdims=True)
    lse = jnp.log(jnp.sum(jnp.exp(logits - m), axis=1, keepdims=True)) + m
    out_ref[...] = logits - lse


_tc_mlp = pl.pallas_call(
    _tc_body,
    grid=(B // BM,),
    in_specs=[
        pl.BlockSpec(memory_space=pltpu.SMEM),
        pl.BlockSpec((BM, SYN), lambda i: (i, 0)),
        pl.BlockSpec((NC, 3, BM, SEM), lambda i: (0, 0, i, 0)),
        pl.BlockSpec((BM, SEM), lambda i: (i, 0)),
        pl.BlockSpec((BM, SEM), lambda i: (i, 0)),
        pl.BlockSpec((BM, SEM), lambda i: (i, 0)),
        pl.BlockSpec((BM, 8), lambda i: (i, 0)),
        pl.BlockSpec((HID, SYN), lambda i: (0, 0)),
        pl.BlockSpec((HID, SEM), lambda i: (0, 0)),
        pl.BlockSpec((HID, SEM), lambda i: (0, 0)),
        pl.BlockSpec((HID, SEM), lambda i: (0, 0)),
        pl.BlockSpec((HID, 8), lambda i: (0, 0)),
        pl.BlockSpec((1, HID), lambda i: (0, 0)),
        pl.BlockSpec((OUT, HID), lambda i: (0, 0)),
        pl.BlockSpec((1, OUT), lambda i: (0, 0)),
    ],
    out_specs=pl.BlockSpec((BM, OUT), lambda i: (i, 0)),
    out_shape=jax.ShapeDtypeStruct((B, OUT), jnp.float32),
    compiler_params=pltpu.CompilerParams(
        dimension_semantics=("arbitrary",)),
)


def kernel(d_onehot, cat_b_ix, hvb_rows, hvb_cols, hvb_vals, hvb_top,
           hvf_rows, hvf_cols, hvf_vals, hvf_top,
           hva_rows, hva_cols, hva_vals, hva_top,
           nullA, use_gpu, ablate_sem,
           cat_embeds, hvec_embeds, W1, b1, W2, b2):
    del use_gpu
    i32, f32 = jnp.int32, jnp.float32
    cols3 = jnp.stack([hvb_cols, hvf_cols, hva_cols]).astype(i32) \
        .reshape(3, NNZ // CHUNK, CHUNK)
    rows3 = jnp.stack([hvb_rows, hvf_rows, hva_rows]).astype(i32) \
        .reshape(3, NNZ // CHUNK, CHUNK)
    vals3 = jnp.stack([hvb_vals, hvf_vals, hva_vals]).astype(f32) \
        .reshape(3, NNZ // CHUNK, CHUNK)
    catix = cat_b_ix.astype(i32).reshape(NW, CAT_PER_W)

    parts, cat_be = _sc_spmm_prog()(cols3, rows3, vals3, catix,
                                    hvec_embeds.astype(f32),
                                    cat_embeds.astype(f32))

    scale = jnp.where(jnp.asarray(ablate_sem) != 0, 0.0, 1.0) \
        .astype(f32).reshape(1, 1)
    small = jnp.concatenate([nullA[:, None], d_onehot], axis=1).astype(f32)
    w1c = W1[:, :SYN]
    w1b = W1[:, SYN:SYN + SEM]
    w1f = W1[:, SYN + SEM:SYN + 2 * SEM]
    w1a = W1[:, SYN + 2 * SEM:SYN + 3 * SEM]
    w1s = W1[:, SYN + 3 * SEM:]
    return _tc_mlp(scale, cat_be, parts, hvb_top, hvf_top, hva_top, small,
                   w1c, w1b, w1f, w1a, w1s,
                   b1.reshape(1, HID), W2, b2.reshape(1, OUT))
